# baseline (speedup 1.0000x reference)
import functools

import jax
import jax.numpy as jnp
from jax import lax
from jax.experimental import pallas as pl
from jax.experimental.pallas import tpu as pltpu
from jax.experimental.pallas import tpu_sc as plsc

_B = 16384
_C = 1000
_NC = 2
_NS = 16
_NW = _NC * _NS
_RPW = _B // _NW
_HALF = _RPW // 2
_NG = _HALF // 16


def _ln(x):
    bits = lax.bitcast_convert_type(x, jnp.int32)
    e = ((bits >> 23) & 0xFF) - 127
    m = lax.bitcast_convert_type(
        (bits & 0x007FFFFF) | 0x3F800000, jnp.float32)
    big = m >= 1.4142135
    m = jnp.where(big, m * 0.5, m)
    e = (e + jnp.where(big, 1, 0)).astype(jnp.float32)
    z = m - 1.0
    w = z / (2.0 + z)
    w2 = w * w
    p = w * (2.0 + w2 * (0.66666667 + w2 * (0.4 + w2 * 0.28571429)))
    return e * 0.69314718 + p


def _sc_body(xt_mat, tgt, psum_out, t_v, big_v, zbuf_v, x0r_v, tmp_v, out_v,
             sem):
    wid = lax.axis_index("s") * _NC + lax.axis_index("c")
    cstart = wid * _RPW
    lanes = lax.iota(jnp.int32, 16)

    pltpu.sync_copy(tgt.at[pl.ds(cstart, _RPW)], t_v)
    pltpu.sync_copy(xt_mat.at[0, pl.ds(cstart, _RPW)], x0r_v)

    acc_tot = jnp.zeros((16,), jnp.float32)
    for h in range(2):
        base = cstart + h * _HALF
        ho = h * _HALF

        gathers = []
        for c in range(_HALF // 128):
            iwin = pl.multiple_of(base + c * 128, 128)
            gathers.append(pltpu.async_copy(
                xt_mat.at[:, pl.ds(iwin, 128)]
                .at[t_v.at[pl.ds(ho + c * 128, 128)]],
                big_v.at[pl.ds(c * 128, 128)], sem))
        for cp in gathers:
            cp.wait()

        def group(g, acc):
            gb = g * 16
            t16 = t_v[pl.ds(ho + gb, 16)]
            off0 = (base + gb) - ((base + gb) // 128) * 128
            xt16 = plsc.load_gather(big_v, [gb + lanes, off0 + lanes])
            e016 = jnp.exp(x0r_v[pl.ds(ho + gb, 16)])
            tmp_v[...] = e016
            nzero = plsc.all_reduce_population_count(t16 == 0)[0]

            @pl.when(nzero > 0)
            def _():
                def lane(l, carry2):
                    t_l = jnp.sum(jnp.where(lanes == l, t16, 0))

                    @pl.when(t_l == 0)
                    def _zcol():
                        i_col = base + gb + l
                        i_al = pl.multiple_of((i_col // 128) * 128, 128)
                        colv = jnp.full((16,), i_col - i_al, jnp.int32)

                        def chunk(q, a):
                            r0 = (q // 16) * 256

                            @pl.when(q % 16 == 0)
                            def _fetch():
                                pltpu.sync_copy(
                                    xt_mat.at[pl.ds(r0, 256),
                                              pl.ds(i_al, 128)],
                                    zbuf_v)

                            vals = plsc.load_gather(
                                zbuf_v, [(q % 16) * 16 + lanes, colv])
                            return a + jnp.exp(vals)

                        zacc = lax.fori_loop(0, 48, chunk,
                                             jnp.zeros((16,), jnp.float32))
                        pltpu.sync_copy(
                            xt_mat.at[pl.ds(768, 232), pl.ds(i_al, 128)],
                            zbuf_v.at[pl.ds(0, 232)])

                        def chunk2(q, a):
                            vals = plsc.load_gather(
                                zbuf_v, [q * 16 + lanes, colv])
                            return a + jnp.exp(vals)

                        zacc2 = lax.fori_loop(0, 14, chunk2, zacc)
                        tidx = jnp.minimum(224 + lanes, 231)
                        tvals = plsc.load_gather(zbuf_v, [tidx, colv])
                        zacc3 = zacc2 + jnp.where(
                            lanes < 8, jnp.exp(tvals), 0.0)
                        s = jnp.sum(zacc3)
                        cur = tmp_v[...]
                        e0_l = jnp.sum(jnp.where(lanes == l, cur, 0.0))
                        tmp_v[...] = jnp.where(lanes == l, s - e0_l, cur)

                    return carry2

                lax.fori_loop(0, 16, lane, 0)

            neg16 = tmp_v[...]
            li16 = _ln(neg16 + jnp.exp(xt16)) - xt16
            return acc + li16

        acc_tot = lax.fori_loop(0, _NG, group, acc_tot)

    sfull = jnp.full((16,), jnp.sum(acc_tot), jnp.float32)
    for q in range(8):
        out_v[pl.ds(q * 16, 16)] = sfull
    pltpu.sync_copy(out_v, psum_out.at[wid])


def _fin_body(ps_ref, out_ref):
    ps = ps_ref[...]
    col = lax.broadcasted_iota(jnp.int32, ps.shape, 1)
    out_ref[0, 0] = jnp.sum(jnp.where(col == 0, ps, 0.0)) / _B


@jax.jit
def kernel(inputs, targets):
    t32 = targets.astype(jnp.int32)
    xt_mat = inputs.T

    mesh = plsc.VectorSubcoreMesh(core_axis_name="c", subcore_axis_name="s",
                                  num_cores=_NC, num_subcores=_NS)
    sc_fn = pl.kernel(
        _sc_body,
        out_type=jax.ShapeDtypeStruct((_NW, 128), jnp.float32),
        mesh=mesh,
        compiler_params=pltpu.CompilerParams(needs_layout_passes=False),
        scratch_types=[
            pltpu.VMEM((_RPW,), jnp.int32),
            pltpu.VMEM((_HALF, 128), jnp.float32),
            pltpu.VMEM((256, 128), jnp.float32),
            pltpu.VMEM((_RPW,), jnp.float32),
            pltpu.VMEM((16,), jnp.float32),
            pltpu.VMEM((128,), jnp.float32),
            pltpu.SemaphoreType.DMA,
        ],
    )
    psum = sc_fn(xt_mat, t32)

    loss = pl.pallas_call(
        _fin_body,
        out_specs=pl.BlockSpec(memory_space=pltpu.SMEM),
        out_shape=jax.ShapeDtypeStruct((1, 1), jnp.float32),
    )(psum)
    return (loss[0, 0]).astype(inputs.dtype)
